# Initial kernel scaffold; baseline (speedup 1.0000x reference)
import jax
import jax.numpy as jnp
from jax.experimental import pallas as pl


def kernel(X_h, edge_index, edge_attr_h, params):
    raise NotImplementedError("write your pallas kernel here")


# speedup vs baseline: 2.0709x; 2.0709x over previous
import functools

import jax
import jax.numpy as jnp
from jax import lax
from jax.experimental import pallas as pl
from jax.experimental.pallas import tpu as pltpu
from jax.experimental.pallas import tpu_sc as plsc

_NC, _NS = 2, 16
_NW = _NC * _NS
_CH = 128
_BE = 1024
_BN = 2000


def _ln(x, g, b):
    m = jnp.mean(x, -1, keepdims=True)
    xc = x - m
    v = jnp.mean(xc * xc, -1, keepdims=True)
    return xc * lax.rsqrt(v + 1e-5) * g + b


def _make_gather(e_pad, d):
    per = e_pad // _NW
    steps = per // _CH
    mesh = plsc.VectorSubcoreMesh(core_axis_name="c", subcore_axis_name="s")

    @functools.partial(
        pl.kernel,
        mesh=mesh,
        compiler_params=pltpu.CompilerParams(use_tc_tiling_on_sc=False),
        out_type=[jax.ShapeDtypeStruct((e_pad, d), jnp.float32),
                  jax.ShapeDtypeStruct((e_pad, d), jnp.float32)],
        scratch_types=[pltpu.VMEM((_CH,), jnp.int32),
                       pltpu.VMEM((_CH,), jnp.int32),
                       pltpu.VMEM((_CH, d), jnp.float32),
                       pltpu.VMEM((_CH, d), jnp.float32),
                       pltpu.SemaphoreType.DMA,
                       pltpu.SemaphoreType.DMA],
    )
    def gather_k(x_hbm, row_hbm, col_hbm, grow_hbm, gcol_hbm,
                 idx_r, idx_c, buf_r, buf_c, sem_r, sem_c):
        wid = lax.axis_index("s") * _NC + lax.axis_index("c")
        base = wid * per

        def step(i, carry):
            off = base + i * _CH
            pltpu.sync_copy(row_hbm.at[pl.ds(off, _CH)], idx_r)
            pltpu.sync_copy(col_hbm.at[pl.ds(off, _CH)], idx_c)
            cr = pltpu.async_copy(x_hbm.at[idx_r], buf_r, sem_r)
            cc = pltpu.async_copy(x_hbm.at[idx_c], buf_c, sem_c)
            cr.wait()
            cc.wait()
            pltpu.sync_copy(buf_r, grow_hbm.at[pl.ds(off, _CH)])
            pltpu.sync_copy(buf_c, gcol_hbm.at[pl.ds(off, _CH)])
            return carry

        lax.fori_loop(0, steps, step, 0)

    return gather_k


def _make_scatter(e_pad, d, n_acc):
    per = e_pad // _NW
    steps = per // _CH
    rows_t = n_acc // _NS
    mesh = plsc.VectorSubcoreMesh(core_axis_name="c", subcore_axis_name="s")

    n_z = rows_t // _CH + 1
    z_last = rows_t - (n_z - 1) * _CH
    scratch = [pltpu.VMEM((_CH,), jnp.int32),
               pltpu.VMEM((_CH, d), jnp.float32),
               pltpu.VMEM((_CH, 16), jnp.float32),
               pltpu.VMEM((_CH, 16), jnp.float32),
               pltpu.VMEM_SHARED((n_acc, d), jnp.float32),
               pltpu.VMEM_SHARED((n_acc, 16), jnp.float32)]

    @functools.partial(
        pl.kernel,
        mesh=mesh,
        compiler_params=pltpu.CompilerParams(use_tc_tiling_on_sc=False),
        out_type=[jax.ShapeDtypeStruct((n_acc, d), jnp.float32),
                  jax.ShapeDtypeStruct((n_acc, d), jnp.float32),
                  jax.ShapeDtypeStruct((n_acc, 16), jnp.float32),
                  jax.ShapeDtypeStruct((n_acc, 16), jnp.float32)],
        scratch_types=scratch,
    )
    def scatter_k(m_hbm, col_hbm, s0_hbm, s1_hbm, c0_hbm, c1_hbm,
                  idx, vals, ones, z16, acc_s, acc_c):
        cid = lax.axis_index("c")
        sid = lax.axis_index("s")
        wid = sid * _NC + cid
        zero16 = jnp.zeros((16,), jnp.float32)
        one16 = jnp.ones((16,), jnp.float32)

        def zrow(i, carry):
            for k in range(d // 16):
                vals[i, pl.ds(k * 16, 16)] = zero16
            z16[i] = zero16
            ones[i] = one16
            return carry

        lax.fori_loop(0, _CH, zrow, 0)

        r0 = sid * rows_t
        for k in range(n_z):
            w = _CH if k < n_z - 1 else z_last
            if w > 0:
                pltpu.sync_copy(vals.at[pl.ds(0, w)],
                                acc_s.at[pl.ds(r0 + k * _CH, w)])
                pltpu.sync_copy(z16.at[pl.ds(0, w)],
                                acc_c.at[pl.ds(r0 + k * _CH, w)])
        plsc.subcore_barrier()

        base = wid * per

        def step(i, carry):
            off = base + i * _CH
            pltpu.sync_copy(col_hbm.at[pl.ds(off, _CH)], idx)
            pltpu.sync_copy(m_hbm.at[pl.ds(off, _CH)], vals)
            pltpu.sync_copy(vals, acc_s.at[idx], add=True)
            pltpu.sync_copy(ones, acc_c.at[idx], add=True)
            return carry

        lax.fori_loop(0, steps, step, 0)
        plsc.subcore_barrier()

        for k in range(n_z):
            w = _CH if k < n_z - 1 else z_last
            if w > 0:
                rr = r0 + k * _CH
                pltpu.sync_copy(acc_s.at[pl.ds(rr, w)], vals.at[pl.ds(0, w)])
                pltpu.sync_copy(acc_c.at[pl.ds(rr, w)], z16.at[pl.ds(0, w)])

                @pl.when(cid == 0)
                def _():
                    pltpu.sync_copy(vals.at[pl.ds(0, w)],
                                    s0_hbm.at[pl.ds(rr, w)])
                    pltpu.sync_copy(z16.at[pl.ds(0, w)],
                                    c0_hbm.at[pl.ds(rr, w)])

                @pl.when(cid == 1)
                def _():
                    pltpu.sync_copy(vals.at[pl.ds(0, w)],
                                    s1_hbm.at[pl.ds(rr, w)])
                    pltpu.sync_copy(z16.at[pl.ds(0, w)],
                                    c1_hbm.at[pl.ds(rr, w)])

    return scatter_k


def _edge_body(grow_ref, gcol_ref, ea_ref, w_ref, c_ref, eout_ref, mout_ref):
    src = grow_ref[...]
    dst = gcol_ref[...]
    ea = ea_ref[...]
    c = c_ref[...]

    def dot(a, wi):
        return jnp.dot(a, w_ref[wi], preferred_element_type=jnp.float32)

    z = dot(src, 0) + dot(dst, 1) + dot(ea, 2) + c[0:1]
    h = _ln(jnp.maximum(z, 0.0), c[1:2], c[2:3])
    h = _ln(jnp.maximum(dot(h, 3) + c[3:4], 0.0), c[4:5], c[5:6])
    e_new = _ln(dot(h, 4) + c[6:7], c[7:8], c[8:9]) + ea
    eout_ref[...] = e_new
    zm = dot(src, 5) + dot(e_new, 6) + c[9:10]
    mout_ref[...] = _ln(jnp.maximum(zm, 0.0), c[10:11], c[11:12])


def _make_edge(e_pad, d):
    grid = (e_pad // _BE,)
    blk = lambda i: (i, 0)
    fixed3 = lambda i: (0, 0, 0)
    fixed2 = lambda i: (0, 0)
    return pl.pallas_call(
        _edge_body,
        grid=grid,
        in_specs=[pl.BlockSpec((_BE, d), blk),
                  pl.BlockSpec((_BE, d), blk),
                  pl.BlockSpec((_BE, d), blk),
                  pl.BlockSpec((7, d, d), fixed3),
                  pl.BlockSpec((12, d), fixed2)],
        out_specs=[pl.BlockSpec((_BE, d), blk),
                   pl.BlockSpec((_BE, d), blk)],
        out_shape=[jax.ShapeDtypeStruct((e_pad, d), jnp.float32),
                   jax.ShapeDtypeStruct((e_pad, d), jnp.float32)],
    )


def _node_body(x_ref, s0_ref, s1_ref, c0_ref, c1_ref, w_ref, c_ref, out_ref):
    x = x_ref[...]
    s = s0_ref[...] + s1_ref[...]
    cnt = c0_ref[...] + c1_ref[...]
    agg = s / jnp.maximum(cnt[:, 0:1], 1.0)
    c = c_ref[...]

    def dot(a, wi):
        return jnp.dot(a, w_ref[wi], preferred_element_type=jnp.float32)

    z = dot(x, 0) + dot(agg, 1) + c[0:1]
    h = _ln(jnp.maximum(z, 0.0), c[1:2], c[2:3])
    out_ref[...] = _ln(dot(h, 2) + c[3:4], c[4:5], c[5:6]) + x


def _make_node(n, d, n_acc):
    grid = (n // _BN,)
    blk = lambda i: (i, 0)
    fixed3 = lambda i: (0, 0, 0)
    fixed2 = lambda i: (0, 0)
    return pl.pallas_call(
        _node_body,
        grid=grid,
        in_specs=[pl.BlockSpec((_BN, d), blk),
                  pl.BlockSpec((_BN, d), blk),
                  pl.BlockSpec((_BN, d), blk),
                  pl.BlockSpec((_BN, 16), blk),
                  pl.BlockSpec((_BN, 16), blk),
                  pl.BlockSpec((3, d, d), fixed3),
                  pl.BlockSpec((6, d), fixed2)],
        out_specs=pl.BlockSpec((_BN, d), blk),
        out_shape=jax.ShapeDtypeStruct((n, d), jnp.float32),
    )


def kernel(X_h, edge_index, edge_attr_h, params):
    n, d = X_h.shape
    e = edge_index.shape[1]
    chunk = _NW * _CH
    e_pad = ((e + chunk - 1) // chunk) * chunk
    n_acc = ((n + 1 + _NS * 8 - 1) // (_NS * 8)) * (_NS * 8)
    pad = e_pad - e

    row = edge_index[0].astype(jnp.int32)
    col = edge_index[1].astype(jnp.int32)
    row_g = jnp.concatenate([row, jnp.zeros((pad,), jnp.int32)])
    col_g = jnp.concatenate([col, jnp.zeros((pad,), jnp.int32)])
    col_s = jnp.concatenate([col, jnp.full((pad,), n, jnp.int32)])
    ea_pad = jnp.concatenate(
        [edge_attr_h, jnp.zeros((pad, d), jnp.float32)], axis=0)

    gather_f = _make_gather(e_pad, d)
    scatter_f = _make_scatter(e_pad, d, n_acc)
    edge_f = _make_edge(e_pad, d)
    node_f = _make_node(n, d, n_acc)

    def fold_edge_params(pe, pn):
        w0 = pe[0]["W"]
        w = jnp.stack([w0[0:d], w0[d:2 * d], w0[2 * d:3 * d],
                       pe[1]["W"], pe[2]["W"],
                       pn[0]["W"][0:d], pn[0]["W"][d:2 * d]])
        c = jnp.stack([pe[0]["b"] + w0[3 * d], pe[0]["g"], pe[0]["beta"],
                       pe[1]["b"], pe[1]["g"], pe[1]["beta"],
                       pe[2]["b"], pe[2]["g"], pe[2]["beta"],
                       pn[0]["b"], pn[0]["g"], pn[0]["beta"]])
        return w, c

    def fold_node_params(pn):
        w1 = pn[1]["W"]
        w = jnp.stack([w1[0:d], w1[d:2 * d], pn[2]["W"]])
        c = jnp.stack([pn[1]["b"] + w1[2 * d], pn[1]["g"], pn[1]["beta"],
                       pn[2]["b"], pn[2]["g"], pn[2]["beta"]])
        return w, c

    def gn_layer(x, ea, pe, pn):
        we, ce = fold_edge_params(pe, pn)
        wn, cn = fold_node_params(pn)
        grow, gcol = gather_f(x, row_g, col_g)
        e_new, m = edge_f(grow, gcol, ea, we, ce)
        s0, s1, c0, c1 = scatter_f(m, col_s)
        x_new = node_f(x, s0, s1, c0, c1, wn, cn)
        return x_new, e_new

    x1, ea1 = gn_layer(X_h, ea_pad, params["gn1_edge"], params["gn1_node"])
    x2, ea2 = gn_layer(x1, ea1, params["gn2_edge"], params["gn2_node"])
    return (x2, ea2[:e], jnp.ones((1, 1), jnp.float32))
